# baseline (speedup 1.0000x reference)
import functools

import jax
import jax.numpy as jnp
from jax import lax
from jax.experimental import pallas as pl
from jax.experimental.pallas import tpu as pltpu
from jax.experimental.pallas import tpu_sc as plsc

TARGET_DISC = 1000
SPEED_DISC = 1000
MAX_ROAD_OPTIONS = 10
EMB_DIM = 8
BATCH = 16384

_NROWS = 2 * TARGET_DISC + SPEED_DISC + MAX_ROAD_OPTIONS

_INFO = plsc.get_sparse_core_info()
_NC, _NS, _L = _INFO.num_cores, _INFO.num_subcores, _INFO.num_lanes
_NW = _NC * _NS
_BPW = BATCH // _NW
_GPW = 4 * _BPW
_NVEC = _BPW // _L


def _searchsorted_right(xv, bins_ref, nbins, lo, inv_step):
    t = jnp.clip((xv - lo) * inv_step, -1.0, float(nbins)) + 1.0
    g = jnp.clip(lax.convert_element_type(t, jnp.int32), 0, nbins)
    bin_hi = plsc.load_gather(bins_ref, [jnp.clip(g, 0, nbins - 1)])
    bin_lo = plsc.load_gather(bins_ref, [jnp.clip(g - 1, 0, nbins - 1)])
    one = jnp.full((_L,), 1, jnp.int32)
    zero = jnp.full((_L,), 0, jnp.int32)
    up = jnp.where((g < nbins) & (bin_hi <= xv), one, zero)
    dn = jnp.where((g > 0) & (bin_lo > xv), one, zero)
    return g + up - dn


def _sc_body(metrics_hbm, table_hbm, tbins_hbm, sbins_hbm, out_hbm,
             m_v, tab_v, tbins_v, sbins_v, idx_v, rows_v):
    wid = lax.axis_index("s") * _NC + lax.axis_index("c")
    base = wid * _BPW

    with jax.named_scope("in_dma"):
        pltpu.sync_copy(metrics_hbm.at[pl.ds(base, _BPW)], m_v)
        pltpu.sync_copy(tbins_hbm, tbins_v)
        pltpu.sync_copy(sbins_hbm, sbins_v)
        pltpu.sync_copy(table_hbm, tab_v)

    lane = lax.iota(jnp.int32, _L)
    with jax.named_scope("idx_compute"):
        for i in range(_NVEC):
            rid = i * _L + lane
            c0 = jnp.full((_L,), 0, jnp.int32)
            x = plsc.load_gather(m_v, [rid, c0])
            y = plsc.load_gather(m_v, [rid, c0 + 1])
            s = plsc.load_gather(m_v, [rid, c0 + 2])
            r = plsc.load_gather(m_v, [rid, c0 + 3])

            xi = jnp.clip(
                _searchsorted_right(x, tbins_v, TARGET_DISC, -0.001, 499500.0),
                0, TARGET_DISC - 1)
            yi = jnp.clip(
                _searchsorted_right(y, tbins_v, TARGET_DISC, -0.001, 499500.0),
                0, TARGET_DISC - 1) + TARGET_DISC
            si = jnp.clip(
                _searchsorted_right(s, sbins_v, SPEED_DISC, -60.0, 8.325),
                0, SPEED_DISC - 1) + 2 * TARGET_DISC
            ri = jnp.clip(lax.convert_element_type(r, jnp.int32),
                          0, MAX_ROAD_OPTIONS - 1) + 2 * TARGET_DISC + SPEED_DISC

            colb = 4 * _L * i + 4 * lane
            plsc.store_scatter(idx_v, [colb], xi)
            plsc.store_scatter(idx_v, [colb + 1], yi)
            plsc.store_scatter(idx_v, [colb + 2], si)
            plsc.store_scatter(idx_v, [colb + 3], ri)

    half = jnp.where(lane < EMB_DIM, jnp.full((_L,), 0, jnp.int32),
                     jnp.full((_L,), 1, jnp.int32))
    lane8 = lane - EMB_DIM * half

    with jax.named_scope("gather"):
        @pl.loop(0, _GPW // 2, unroll=8)
        def _gather(k):
            rid = plsc.load_gather(idx_v, [2 * k + half])
            val = plsc.load_gather(tab_v, [rid, lane8])
            rows_v[pl.ds(_L * k, _L)] = val

    with jax.named_scope("out_dma"):
        pltpu.sync_copy(rows_v, out_hbm.at[pl.ds(wid * _GPW * EMB_DIM,
                                                 _GPW * EMB_DIM)])


_sc_lookup = functools.partial(
    pl.kernel,
    out_type=jax.ShapeDtypeStruct((BATCH * 4 * EMB_DIM,), jnp.float32),
    mesh=plsc.VectorSubcoreMesh(core_axis_name="c", subcore_axis_name="s"),
    compiler_params=pltpu.CompilerParams(
        needs_layout_passes=False, use_tc_tiling_on_sc=False),
    scratch_types=[
        pltpu.VMEM((_BPW, 4), jnp.float32),
        pltpu.VMEM((_NROWS, EMB_DIM), jnp.float32),
        pltpu.VMEM((TARGET_DISC,), jnp.float32),
        pltpu.VMEM((SPEED_DISC,), jnp.float32),
        pltpu.VMEM((_GPW,), jnp.int32),
        pltpu.VMEM((_GPW * EMB_DIM,), jnp.float32),
    ],
)(_sc_body)


def kernel(metrics, target_x_emb, target_y_emb, speed_emb, road_option_emb):
    table = jnp.concatenate(
        [target_x_emb, target_y_emb, speed_emb, road_option_emb], axis=0)
    tbins = jnp.linspace(-0.001, 0.001, TARGET_DISC).astype(jnp.float32)
    sbins = jnp.linspace(-60.0, 60.0, SPEED_DISC).astype(jnp.float32)
    out = _sc_lookup(metrics, table, tbins, sbins)
    out = out.reshape(BATCH, 4 * EMB_DIM)
    return (out, out)


# speedup vs baseline: 1.0948x; 1.0948x over previous
import functools

import jax
import jax.numpy as jnp
from jax import lax
from jax.experimental import pallas as pl
from jax.experimental.pallas import tpu as pltpu
from jax.experimental.pallas import tpu_sc as plsc

TARGET_DISC = 1000
SPEED_DISC = 1000
MAX_ROAD_OPTIONS = 10
EMB_DIM = 8
BATCH = 16384

_NROWS = 2 * TARGET_DISC + SPEED_DISC + MAX_ROAD_OPTIONS

_INFO = plsc.get_sparse_core_info()
_NC, _NS, _L = _INFO.num_cores, _INFO.num_subcores, _INFO.num_lanes
_NW = _NC * _NS
_BPW = BATCH // _NW
_GPW = 4 * _BPW
_NVEC = _BPW // _L


def _searchsorted_right(xv, bins_ref, nbins, lo, inv_step):
    t = jnp.clip((xv - lo) * inv_step, -1.0, float(nbins)) + 1.0
    g = jnp.clip(lax.convert_element_type(t, jnp.int32), 0, nbins)
    bin_hi = plsc.load_gather(bins_ref, [jnp.clip(g, 0, nbins - 1)])
    bin_lo = plsc.load_gather(bins_ref, [jnp.clip(g - 1, 0, nbins - 1)])
    one = jnp.full((_L,), 1, jnp.int32)
    zero = jnp.full((_L,), 0, jnp.int32)
    up = jnp.where((g < nbins) & (bin_hi <= xv), one, zero)
    dn = jnp.where((g > 0) & (bin_lo > xv), one, zero)
    return g + up - dn


def _sc_body(metrics_hbm, tx_hbm, ty_hbm, sp_hbm, ro_hbm,
             tbins_hbm, sbins_hbm, out_hbm,
             m_v, tab_v, tbins_v, sbins_v, idx_v, rows_v):
    wid = lax.axis_index("s") * _NC + lax.axis_index("c")
    base = wid * _BPW

    with jax.named_scope("in_dma"):
        pltpu.sync_copy(metrics_hbm.at[pl.ds(base, _BPW)], m_v)
        pltpu.sync_copy(tbins_hbm, tbins_v)
        pltpu.sync_copy(sbins_hbm, sbins_v)
        pltpu.sync_copy(tx_hbm, tab_v.at[pl.ds(0, TARGET_DISC)])
        pltpu.sync_copy(ty_hbm, tab_v.at[pl.ds(TARGET_DISC, TARGET_DISC)])
        pltpu.sync_copy(sp_hbm, tab_v.at[pl.ds(2 * TARGET_DISC, SPEED_DISC)])
        pltpu.sync_copy(
            ro_hbm,
            tab_v.at[pl.ds(2 * TARGET_DISC + SPEED_DISC, MAX_ROAD_OPTIONS)])

    lane = lax.iota(jnp.int32, _L)
    with jax.named_scope("idx_compute"):
        for i in range(_NVEC):
            rid = i * _L + lane
            c0 = jnp.full((_L,), 0, jnp.int32)
            x = plsc.load_gather(m_v, [rid, c0])
            y = plsc.load_gather(m_v, [rid, c0 + 1])
            s = plsc.load_gather(m_v, [rid, c0 + 2])
            r = plsc.load_gather(m_v, [rid, c0 + 3])

            xi = jnp.clip(
                _searchsorted_right(x, tbins_v, TARGET_DISC, -0.001, 499500.0),
                0, TARGET_DISC - 1)
            yi = jnp.clip(
                _searchsorted_right(y, tbins_v, TARGET_DISC, -0.001, 499500.0),
                0, TARGET_DISC - 1) + TARGET_DISC
            si = jnp.clip(
                _searchsorted_right(s, sbins_v, SPEED_DISC, -60.0, 8.325),
                0, SPEED_DISC - 1) + 2 * TARGET_DISC
            ri = jnp.clip(lax.convert_element_type(r, jnp.int32),
                          0, MAX_ROAD_OPTIONS - 1) + 2 * TARGET_DISC + SPEED_DISC

            colb = 4 * _L * i + 4 * lane
            plsc.store_scatter(idx_v, [colb], xi)
            plsc.store_scatter(idx_v, [colb + 1], yi)
            plsc.store_scatter(idx_v, [colb + 2], si)
            plsc.store_scatter(idx_v, [colb + 3], ri)

    half = jnp.where(lane < EMB_DIM, jnp.full((_L,), 0, jnp.int32),
                     jnp.full((_L,), 1, jnp.int32))
    lane8 = lane - EMB_DIM * half

    with jax.named_scope("gather"):
        @plsc.parallel_loop(0, _BPW, unroll=8)
        def _gather(b):
            rid0 = plsc.load_gather(idx_v, [4 * b + half])
            val0 = plsc.load_gather(tab_v, [rid0, lane8])
            rows_v[b, pl.ds(0, _L)] = val0
            rid1 = plsc.load_gather(idx_v, [4 * b + 2 + half])
            val1 = plsc.load_gather(tab_v, [rid1, lane8])
            rows_v[b, pl.ds(_L, _L)] = val1

    with jax.named_scope("out_dma"):
        pltpu.sync_copy(rows_v, out_hbm.at[pl.ds(base, _BPW)])


_sc_lookup = functools.partial(
    pl.kernel,
    out_type=jax.ShapeDtypeStruct((BATCH, 4 * EMB_DIM), jnp.float32),
    mesh=plsc.VectorSubcoreMesh(core_axis_name="c", subcore_axis_name="s"),
    compiler_params=pltpu.CompilerParams(
        needs_layout_passes=False, use_tc_tiling_on_sc=False),
    scratch_types=[
        pltpu.VMEM((_BPW, 4), jnp.float32),
        pltpu.VMEM((_NROWS, EMB_DIM), jnp.float32),
        pltpu.VMEM((TARGET_DISC,), jnp.float32),
        pltpu.VMEM((SPEED_DISC,), jnp.float32),
        pltpu.VMEM((_GPW,), jnp.int32),
        pltpu.VMEM((_BPW, 4 * EMB_DIM), jnp.float32),
    ],
)(_sc_body)


def kernel(metrics, target_x_emb, target_y_emb, speed_emb, road_option_emb):
    tbins = jnp.linspace(-0.001, 0.001, TARGET_DISC).astype(jnp.float32)
    sbins = jnp.linspace(-60.0, 60.0, SPEED_DISC).astype(jnp.float32)
    out = _sc_lookup(metrics, target_x_emb, target_y_emb, speed_emb,
                     road_option_emb, tbins, sbins)
    return (out, out)


# speedup vs baseline: 1.3699x; 1.2513x over previous
import functools

import jax
import jax.numpy as jnp
from jax import lax
from jax.experimental import pallas as pl
from jax.experimental.pallas import tpu as pltpu
from jax.experimental.pallas import tpu_sc as plsc

TARGET_DISC = 1000
SPEED_DISC = 1000
MAX_ROAD_OPTIONS = 10
EMB_DIM = 8
BATCH = 16384

_NROWS = 2 * TARGET_DISC + SPEED_DISC + MAX_ROAD_OPTIONS

_INFO = plsc.get_sparse_core_info()
_NC, _NS, _L = _INFO.num_cores, _INFO.num_subcores, _INFO.num_lanes
_NW = _NC * _NS
_BPW = BATCH // _NW
_GPW = 4 * _BPW
_NVEC = _BPW // _L


def _searchsorted_right(xv, bins_ref, nbins, lo, inv_step):
    t = jnp.clip((xv - lo) * inv_step, -1.0, float(nbins)) + 1.0
    g = jnp.clip(lax.convert_element_type(t, jnp.int32), 0, nbins)
    bin_hi = plsc.load_gather(bins_ref, [jnp.clip(g, 0, nbins - 1)])
    bin_lo = plsc.load_gather(bins_ref, [jnp.clip(g - 1, 0, nbins - 1)])
    one = jnp.full((_L,), 1, jnp.int32)
    zero = jnp.full((_L,), 0, jnp.int32)
    up = jnp.where((g < nbins) & (bin_hi <= xv), one, zero)
    dn = jnp.where((g > 0) & (bin_lo > xv), one, zero)
    return g + up - dn


def _sc_body(xc_hbm, yc_hbm, sc_hbm, rc_hbm, tx_hbm, ty_hbm, sp_hbm, ro_hbm,
             tbins_hbm, sbins_hbm, out_hbm,
             mx_v, my_v, ms_v, mr_v, tab_v, tbins_v, sbins_v, idx_v, rows_v):
    wid = lax.axis_index("s") * _NC + lax.axis_index("c")
    base = wid * _BPW

    with jax.named_scope("in_dma"):
        pltpu.sync_copy(xc_hbm.at[pl.ds(base, _BPW)], mx_v)
        pltpu.sync_copy(yc_hbm.at[pl.ds(base, _BPW)], my_v)
        pltpu.sync_copy(sc_hbm.at[pl.ds(base, _BPW)], ms_v)
        pltpu.sync_copy(rc_hbm.at[pl.ds(base, _BPW)], mr_v)
        pltpu.sync_copy(tbins_hbm, tbins_v)
        pltpu.sync_copy(sbins_hbm, sbins_v)
        pltpu.sync_copy(tx_hbm, tab_v.at[pl.ds(0, TARGET_DISC)])
        pltpu.sync_copy(ty_hbm, tab_v.at[pl.ds(TARGET_DISC, TARGET_DISC)])
        pltpu.sync_copy(sp_hbm, tab_v.at[pl.ds(2 * TARGET_DISC, SPEED_DISC)])
        pltpu.sync_copy(
            ro_hbm,
            tab_v.at[pl.ds(2 * TARGET_DISC + SPEED_DISC, MAX_ROAD_OPTIONS)])

    lane = lax.iota(jnp.int32, _L)
    with jax.named_scope("idx_compute"):
        for i in range(_NVEC):
            x = mx_v[pl.ds(i * _L, _L)]
            y = my_v[pl.ds(i * _L, _L)]
            s = ms_v[pl.ds(i * _L, _L)]
            r = mr_v[pl.ds(i * _L, _L)]

            xi = jnp.clip(
                _searchsorted_right(x, tbins_v, TARGET_DISC, -0.001, 499500.0),
                0, TARGET_DISC - 1)
            yi = jnp.clip(
                _searchsorted_right(y, tbins_v, TARGET_DISC, -0.001, 499500.0),
                0, TARGET_DISC - 1) + TARGET_DISC
            si = jnp.clip(
                _searchsorted_right(s, sbins_v, SPEED_DISC, -60.0, 8.325),
                0, SPEED_DISC - 1) + 2 * TARGET_DISC
            ri = jnp.clip(lax.convert_element_type(r, jnp.int32),
                          0, MAX_ROAD_OPTIONS - 1) + 2 * TARGET_DISC + SPEED_DISC

            colb = 4 * _L * i + 4 * lane
            plsc.store_scatter(idx_v, [colb], xi)
            plsc.store_scatter(idx_v, [colb + 1], yi)
            plsc.store_scatter(idx_v, [colb + 2], si)
            plsc.store_scatter(idx_v, [colb + 3], ri)

    half = jnp.where(lane < EMB_DIM, jnp.full((_L,), 0, jnp.int32),
                     jnp.full((_L,), 1, jnp.int32))
    lane8 = lane - EMB_DIM * half

    with jax.named_scope("gather"):
        @plsc.parallel_loop(0, _BPW, unroll=8)
        def _gather(b):
            rid0 = plsc.load_gather(idx_v, [4 * b + half])
            val0 = plsc.load_gather(tab_v, [rid0, lane8])
            rows_v[b, pl.ds(0, _L)] = val0
            rid1 = plsc.load_gather(idx_v, [4 * b + 2 + half])
            val1 = plsc.load_gather(tab_v, [rid1, lane8])
            rows_v[b, pl.ds(_L, _L)] = val1

    with jax.named_scope("out_dma"):
        pltpu.sync_copy(rows_v, out_hbm.at[pl.ds(base, _BPW)])


_sc_lookup = functools.partial(
    pl.kernel,
    out_type=jax.ShapeDtypeStruct((BATCH, 4 * EMB_DIM), jnp.float32),
    mesh=plsc.VectorSubcoreMesh(core_axis_name="c", subcore_axis_name="s"),
    compiler_params=pltpu.CompilerParams(
        needs_layout_passes=False, use_tc_tiling_on_sc=False),
    scratch_types=[
        pltpu.VMEM((_BPW,), jnp.float32),
        pltpu.VMEM((_BPW,), jnp.float32),
        pltpu.VMEM((_BPW,), jnp.float32),
        pltpu.VMEM((_BPW,), jnp.float32),
        pltpu.VMEM((_NROWS, EMB_DIM), jnp.float32),
        pltpu.VMEM((TARGET_DISC,), jnp.float32),
        pltpu.VMEM((SPEED_DISC,), jnp.float32),
        pltpu.VMEM((_GPW,), jnp.int32),
        pltpu.VMEM((_BPW, 4 * EMB_DIM), jnp.float32),
    ],
)(_sc_body)


def kernel(metrics, target_x_emb, target_y_emb, speed_emb, road_option_emb):
    tbins = jnp.linspace(-0.001, 0.001, TARGET_DISC).astype(jnp.float32)
    sbins = jnp.linspace(-60.0, 60.0, SPEED_DISC).astype(jnp.float32)
    out = _sc_lookup(metrics[:, 0], metrics[:, 1], metrics[:, 2],
                     metrics[:, 3], target_x_emb, target_y_emb, speed_emb,
                     road_option_emb, tbins, sbins)
    return (out, out)


# speedup vs baseline: 1.4768x; 1.0780x over previous
import functools

import jax
import jax.numpy as jnp
from jax import lax
from jax.experimental import pallas as pl
from jax.experimental.pallas import tpu as pltpu
from jax.experimental.pallas import tpu_sc as plsc

TARGET_DISC = 1000
SPEED_DISC = 1000
MAX_ROAD_OPTIONS = 10
EMB_DIM = 8
BATCH = 16384

_NROWS = 2 * TARGET_DISC + SPEED_DISC + MAX_ROAD_OPTIONS

_INFO = plsc.get_sparse_core_info()
_NC, _NS, _L = _INFO.num_cores, _INFO.num_subcores, _INFO.num_lanes
_NW = _NC * _NS
_BPW = BATCH // _NW
_GPW = 4 * _BPW
_NVEC = _BPW // _L


def _searchsorted_right(xv, bins_ref, nbins, lo, inv_step):
    t = jnp.clip((xv - lo) * inv_step, -1.0, float(nbins)) + 1.0
    g = jnp.clip(lax.convert_element_type(t, jnp.int32), 0, nbins)
    bin_hi = plsc.load_gather(bins_ref, [jnp.clip(g, 0, nbins - 1)])
    bin_lo = plsc.load_gather(bins_ref, [jnp.clip(g - 1, 0, nbins - 1)])
    one = jnp.full((_L,), 1, jnp.int32)
    zero = jnp.full((_L,), 0, jnp.int32)
    up = jnp.where((g < nbins) & (bin_hi <= xv), one, zero)
    dn = jnp.where((g > 0) & (bin_lo > xv), one, zero)
    return g + up - dn


def _sc_body(xc_hbm, yc_hbm, sc_hbm, rc_hbm, tx_hbm, ty_hbm, sp_hbm, ro_hbm,
             tbins_hbm, sbins_hbm, out_hbm,
             mx_v, my_v, ms_v, mr_v, tab_sh, tab_v, tbins_v, sbins_v,
             idx_v, rows_v):
    sid = lax.axis_index("s")
    wid = sid * _NC + lax.axis_index("c")
    base = wid * _BPW

    with jax.named_scope("in_dma"):
        @pl.when(sid == 0)
        def _stage():
            pltpu.sync_copy(tx_hbm, tab_sh.at[pl.ds(0, TARGET_DISC)])
            pltpu.sync_copy(ty_hbm, tab_sh.at[pl.ds(TARGET_DISC, TARGET_DISC)])
            pltpu.sync_copy(sp_hbm,
                            tab_sh.at[pl.ds(2 * TARGET_DISC, SPEED_DISC)])
            pltpu.sync_copy(
                ro_hbm,
                tab_sh.at[pl.ds(2 * TARGET_DISC + SPEED_DISC,
                                MAX_ROAD_OPTIONS)])

        pltpu.sync_copy(xc_hbm.at[pl.ds(base, _BPW)], mx_v)
        pltpu.sync_copy(yc_hbm.at[pl.ds(base, _BPW)], my_v)
        pltpu.sync_copy(sc_hbm.at[pl.ds(base, _BPW)], ms_v)
        pltpu.sync_copy(rc_hbm.at[pl.ds(base, _BPW)], mr_v)
        pltpu.sync_copy(tbins_hbm, tbins_v)
        pltpu.sync_copy(sbins_hbm, sbins_v)
        plsc.subcore_barrier()
        pltpu.sync_copy(tab_sh, tab_v)

    lane = lax.iota(jnp.int32, _L)
    with jax.named_scope("idx_compute"):
        for i in range(_NVEC):
            x = mx_v[pl.ds(i * _L, _L)]
            y = my_v[pl.ds(i * _L, _L)]
            s = ms_v[pl.ds(i * _L, _L)]
            r = mr_v[pl.ds(i * _L, _L)]

            xi = jnp.clip(
                _searchsorted_right(x, tbins_v, TARGET_DISC, -0.001, 499500.0),
                0, TARGET_DISC - 1)
            yi = jnp.clip(
                _searchsorted_right(y, tbins_v, TARGET_DISC, -0.001, 499500.0),
                0, TARGET_DISC - 1) + TARGET_DISC
            si = jnp.clip(
                _searchsorted_right(s, sbins_v, SPEED_DISC, -60.0, 8.325),
                0, SPEED_DISC - 1) + 2 * TARGET_DISC
            ri = jnp.clip(lax.convert_element_type(r, jnp.int32),
                          0, MAX_ROAD_OPTIONS - 1) + 2 * TARGET_DISC + SPEED_DISC

            colb = 4 * _L * i + 4 * lane
            plsc.store_scatter(idx_v, [colb], xi)
            plsc.store_scatter(idx_v, [colb + 1], yi)
            plsc.store_scatter(idx_v, [colb + 2], si)
            plsc.store_scatter(idx_v, [colb + 3], ri)

    half = jnp.where(lane < EMB_DIM, jnp.full((_L,), 0, jnp.int32),
                     jnp.full((_L,), 1, jnp.int32))
    lane8 = lane - EMB_DIM * half

    with jax.named_scope("gather"):
        @plsc.parallel_loop(0, _BPW, unroll=8)
        def _gather(b):
            bb = jnp.full((_L,), 0, jnp.int32) + b
            rid0 = plsc.load_gather(idx_v, [4 * b + half])
            val0 = plsc.load_gather(tab_v, [rid0, lane8])
            plsc.store_scatter(rows_v, [lane, bb], val0)
            rid1 = plsc.load_gather(idx_v, [4 * b + 2 + half])
            val1 = plsc.load_gather(tab_v, [rid1, lane8])
            plsc.store_scatter(rows_v, [lane + _L, bb], val1)

    with jax.named_scope("out_dma"):
        pltpu.sync_copy(rows_v,
                        out_hbm.at[pl.ds(0, 4 * EMB_DIM), pl.ds(base, _BPW)])


_sc_lookup = functools.partial(
    pl.kernel,
    out_type=jax.ShapeDtypeStruct((4 * EMB_DIM, BATCH), jnp.float32),
    mesh=plsc.VectorSubcoreMesh(core_axis_name="c", subcore_axis_name="s"),
    compiler_params=pltpu.CompilerParams(
        needs_layout_passes=False, use_tc_tiling_on_sc=False),
    scratch_types=[
        pltpu.VMEM((_BPW,), jnp.float32),
        pltpu.VMEM((_BPW,), jnp.float32),
        pltpu.VMEM((_BPW,), jnp.float32),
        pltpu.VMEM((_BPW,), jnp.float32),
        pltpu.VMEM_SHARED((_NROWS, EMB_DIM), jnp.float32),
        pltpu.VMEM((_NROWS, EMB_DIM), jnp.float32),
        pltpu.VMEM((TARGET_DISC,), jnp.float32),
        pltpu.VMEM((SPEED_DISC,), jnp.float32),
        pltpu.VMEM((_GPW,), jnp.int32),
        pltpu.VMEM((4 * EMB_DIM, _BPW), jnp.float32),
    ],
)(_sc_body)


def kernel(metrics, target_x_emb, target_y_emb, speed_emb, road_option_emb):
    tbins = jnp.linspace(-0.001, 0.001, TARGET_DISC).astype(jnp.float32)
    sbins = jnp.linspace(-60.0, 60.0, SPEED_DISC).astype(jnp.float32)
    out = _sc_lookup(metrics[:, 0], metrics[:, 1], metrics[:, 2],
                     metrics[:, 3], target_x_emb, target_y_emb, speed_emb,
                     road_option_emb, tbins, sbins).T
    return (out, out)


# speedup vs baseline: 1.8262x; 1.2366x over previous
import functools

import jax
import jax.numpy as jnp
from jax import lax
from jax.experimental import pallas as pl
from jax.experimental.pallas import tpu as pltpu
from jax.experimental.pallas import tpu_sc as plsc

TARGET_DISC = 1000
SPEED_DISC = 1000
MAX_ROAD_OPTIONS = 10
EMB_DIM = 8
BATCH = 16384

_NROWS = 2 * TARGET_DISC + SPEED_DISC + MAX_ROAD_OPTIONS

_INFO = plsc.get_sparse_core_info()
_NC, _NS, _L = _INFO.num_cores, _INFO.num_subcores, _INFO.num_lanes
_NW = _NC * _NS
_BPW = BATCH // _NW
_GPW = 4 * _BPW
_NVEC = _BPW // _L


def _searchsorted_right(xv, bins_ref, nbins, lo, inv_step):
    t = jnp.clip((xv - lo) * inv_step, -1.0, float(nbins)) + 1.0
    g = jnp.clip(lax.convert_element_type(t, jnp.int32), 0, nbins)
    bin_hi = plsc.load_gather(bins_ref, [jnp.clip(g, 0, nbins - 1)])
    bin_lo = plsc.load_gather(bins_ref, [jnp.clip(g - 1, 0, nbins - 1)])
    one = jnp.full((_L,), 1, jnp.int32)
    zero = jnp.full((_L,), 0, jnp.int32)
    up = jnp.where((g < nbins) & (bin_hi <= xv), one, zero)
    dn = jnp.where((g > 0) & (bin_lo > xv), one, zero)
    return g + up - dn


def _sc_body(xc_hbm, yc_hbm, sc_hbm, rc_hbm, tx_hbm, ty_hbm, sp_hbm, ro_hbm,
             tbins_hbm, sbins_hbm, out_hbm,
             mx_v, my_v, ms_v, mr_v, tab_v, tbins_v, sbins_v,
             idx_v, rows_v, sem):
    wid = lax.axis_index("s") * _NC + lax.axis_index("c")
    base = wid * _BPW

    with jax.named_scope("in_dma"):
        copies = [
            pltpu.async_copy(xc_hbm.at[pl.ds(base, _BPW)], mx_v, sem),
            pltpu.async_copy(yc_hbm.at[pl.ds(base, _BPW)], my_v, sem),
            pltpu.async_copy(sc_hbm.at[pl.ds(base, _BPW)], ms_v, sem),
            pltpu.async_copy(rc_hbm.at[pl.ds(base, _BPW)], mr_v, sem),
            pltpu.async_copy(tbins_hbm, tbins_v, sem),
            pltpu.async_copy(sbins_hbm, sbins_v, sem),
            pltpu.async_copy(tx_hbm, tab_v.at[pl.ds(0, TARGET_DISC)], sem),
            pltpu.async_copy(ty_hbm, tab_v.at[pl.ds(TARGET_DISC, TARGET_DISC)],
                             sem),
            pltpu.async_copy(sp_hbm,
                             tab_v.at[pl.ds(2 * TARGET_DISC, SPEED_DISC)],
                             sem),
            pltpu.async_copy(
                ro_hbm,
                tab_v.at[pl.ds(2 * TARGET_DISC + SPEED_DISC,
                               MAX_ROAD_OPTIONS)], sem),
        ]
        for c in copies:
            c.wait()

    with jax.named_scope("idx_compute"):
        for i in range(_NVEC):
            x = mx_v[pl.ds(i * _L, _L)]
            y = my_v[pl.ds(i * _L, _L)]
            s = ms_v[pl.ds(i * _L, _L)]
            r = mr_v[pl.ds(i * _L, _L)]

            xi = jnp.clip(
                _searchsorted_right(x, tbins_v, TARGET_DISC, -0.001, 499500.0),
                0, TARGET_DISC - 1)
            yi = jnp.clip(
                _searchsorted_right(y, tbins_v, TARGET_DISC, -0.001, 499500.0),
                0, TARGET_DISC - 1) + TARGET_DISC
            si = jnp.clip(
                _searchsorted_right(s, sbins_v, SPEED_DISC, -60.0, 8.325),
                0, SPEED_DISC - 1) + 2 * TARGET_DISC
            ri = jnp.clip(lax.convert_element_type(r, jnp.int32),
                          0, MAX_ROAD_OPTIONS - 1) + 2 * TARGET_DISC + SPEED_DISC

            sl = pl.ds(i * _L, _L)
            idx_v[0, sl] = xi
            idx_v[1, sl] = yi
            idx_v[2, sl] = si
            idx_v[3, sl] = ri

    with jax.named_scope("gather"):
        @plsc.parallel_loop(0, _NVEC, unroll=4)
        def _gather(j):
            sl = pl.ds(j * _L, _L)
            for c in range(4):
                rid = idx_v[c, sl]
                for e in range(EMB_DIM):
                    col = jnp.full((_L,), e, jnp.int32)
                    rows_v[c * EMB_DIM + e, sl] = plsc.load_gather(
                        tab_v, [rid, col])

    with jax.named_scope("out_dma"):
        pltpu.sync_copy(rows_v,
                        out_hbm.at[pl.ds(0, 4 * EMB_DIM), pl.ds(base, _BPW)])


_sc_lookup = functools.partial(
    pl.kernel,
    out_type=jax.ShapeDtypeStruct((4 * EMB_DIM, BATCH), jnp.float32),
    mesh=plsc.VectorSubcoreMesh(core_axis_name="c", subcore_axis_name="s"),
    compiler_params=pltpu.CompilerParams(
        needs_layout_passes=False, use_tc_tiling_on_sc=False),
    scratch_types=[
        pltpu.VMEM((_BPW,), jnp.float32),
        pltpu.VMEM((_BPW,), jnp.float32),
        pltpu.VMEM((_BPW,), jnp.float32),
        pltpu.VMEM((_BPW,), jnp.float32),
        pltpu.VMEM((_NROWS, EMB_DIM), jnp.float32),
        pltpu.VMEM((TARGET_DISC,), jnp.float32),
        pltpu.VMEM((SPEED_DISC,), jnp.float32),
        pltpu.VMEM((4, _BPW), jnp.int32),
        pltpu.VMEM((4 * EMB_DIM, _BPW), jnp.float32),
        pltpu.SemaphoreType.DMA,
    ],
)(_sc_body)


def kernel(metrics, target_x_emb, target_y_emb, speed_emb, road_option_emb):
    tbins = jnp.linspace(-0.001, 0.001, TARGET_DISC).astype(jnp.float32)
    sbins = jnp.linspace(-60.0, 60.0, SPEED_DISC).astype(jnp.float32)
    out = _sc_lookup(metrics[:, 0], metrics[:, 1], metrics[:, 2],
                     metrics[:, 3], target_x_emb, target_y_emb, speed_emb,
                     road_option_emb, tbins, sbins).T
    return (out, out)


# speedup vs baseline: 2.1282x; 1.1654x over previous
import functools

import numpy as np

import jax
import jax.numpy as jnp
from jax import lax
from jax.experimental import pallas as pl
from jax.experimental.pallas import tpu as pltpu
from jax.experimental.pallas import tpu_sc as plsc

TARGET_DISC = 1000
SPEED_DISC = 1000
MAX_ROAD_OPTIONS = 10
EMB_DIM = 8
BATCH = 16384

_INFO = plsc.get_sparse_core_info()
_NC, _NS, _L = _INFO.num_cores, _INFO.num_subcores, _INFO.num_lanes
_NW = _NC * _NS
_NCOMP = 4
_NGRP = _NW // _NCOMP
_BPW = BATCH // _NGRP
_NVEC = _BPW // _L
_TSEG = 1000


def _f32_linspace(start, stop, num):
    inv = np.float32(np.float32(1.0) / np.float32(num - 1))
    i = np.arange(num - 1, dtype=np.float32)
    t = i * inv
    head = np.float32(start) * (np.float32(1.0) - t) \
        + i * (np.float32(stop) * inv)
    return np.concatenate([head, np.array([stop], np.float32)])


_BINS = np.stack([
    _f32_linspace(-0.001, 0.001, TARGET_DISC),
    _f32_linspace(-0.001, 0.001, TARGET_DISC),
    _f32_linspace(-60.0, 60.0, SPEED_DISC),
    np.zeros(TARGET_DISC, np.float32),
])


def _sc_body(mc_hbm, tab_hbm, bins_hbm, out_hbm,
             m_v, tab_v, bins_v, idx_v, rows_v, sem):
    wid = lax.axis_index("s") * _NC + lax.axis_index("c")
    comp = lax.rem(wid, _NCOMP)
    base = lax.div(wid, _NCOMP) * _BPW

    with jax.named_scope("in_dma"):
        copies = [
            pltpu.async_copy(mc_hbm.at[comp, pl.ds(base, _BPW)], m_v, sem),
            pltpu.async_copy(bins_hbm.at[comp], bins_v, sem),
        ]
        coff = comp * _TSEG
        for k in range(4):
            p = lax.rem(k + wid, 4) * 2
            copies.append(pltpu.async_copy(
                tab_hbm.at[pl.ds(p, 2), pl.ds(coff, _TSEG)],
                tab_v.at[pl.ds(p, 2)], sem))
        for c in copies:
            c.wait()

    is_road = comp == _NCOMP - 1
    lo = jnp.where(comp < 2, jnp.float32(-0.001),
                   jnp.where(comp == 2, jnp.float32(-60.0), jnp.float32(0.0)))
    inv_step = jnp.where(comp < 2, jnp.float32(499500.0),
                         jnp.where(comp == 2, jnp.float32(8.325),
                                   jnp.float32(1.0)))
    hi_clip = jnp.where(is_road, MAX_ROAD_OPTIONS - 1, _TSEG - 1)
    road16 = jnp.broadcast_to(is_road, (_L,))
    nb = _TSEG

    with jax.named_scope("idx_compute"):
        for i in range(_NVEC):
            x = m_v[pl.ds(i * _L, _L)]
            t = jnp.clip((x - lo) * inv_step, -1.0, float(nb)) + 1.0
            g = jnp.clip(lax.convert_element_type(t, jnp.int32), 0, nb)
            bin_hi = plsc.load_gather(bins_v, [jnp.clip(g, 0, nb - 1)])
            bin_lo = plsc.load_gather(bins_v, [jnp.clip(g - 1, 0, nb - 1)])
            one = jnp.full((_L,), 1, jnp.int32)
            zero = jnp.full((_L,), 0, jnp.int32)
            up = jnp.where((g < nb) & (bin_hi <= x), one, zero)
            dn = jnp.where((g > 0) & (bin_lo > x), one, zero)
            srch = g + up - dn
            road = lax.convert_element_type(x, jnp.int32)
            idx_v[pl.ds(i * _L, _L)] = jnp.clip(
                jnp.where(road16, road, srch), 0, hi_clip)

    with jax.named_scope("gather"):
        @plsc.parallel_loop(0, _NVEC, unroll=4)
        def _gather(j):
            sl = pl.ds(j * _L, _L)
            rid = idx_v[sl]
            for e in range(EMB_DIM):
                row = jnp.full((_L,), e, jnp.int32)
                rows_v[e, sl] = plsc.load_gather(tab_v, [row, rid])

    with jax.named_scope("out_dma"):
        pltpu.sync_copy(
            rows_v,
            out_hbm.at[pl.ds(comp * EMB_DIM, EMB_DIM), pl.ds(base, _BPW)])


_sc_lookup = functools.partial(
    pl.kernel,
    out_type=jax.ShapeDtypeStruct((_NCOMP * EMB_DIM, BATCH), jnp.float32),
    mesh=plsc.VectorSubcoreMesh(core_axis_name="c", subcore_axis_name="s"),
    compiler_params=pltpu.CompilerParams(
        needs_layout_passes=False, use_tc_tiling_on_sc=False),
    scratch_types=[
        pltpu.VMEM((_BPW,), jnp.float32),
        pltpu.VMEM((EMB_DIM, _TSEG), jnp.float32),
        pltpu.VMEM((_TSEG,), jnp.float32),
        pltpu.VMEM((_BPW,), jnp.int32),
        pltpu.VMEM((EMB_DIM, _BPW), jnp.float32),
        pltpu.SemaphoreType.DMA,
    ],
)(_sc_body)


def kernel(metrics, target_x_emb, target_y_emb, speed_emb, road_option_emb):
    mcols = jnp.stack([metrics[:, 0], metrics[:, 1],
                       metrics[:, 2], metrics[:, 3]])
    table_t = jnp.concatenate(
        [target_x_emb.T, target_y_emb.T, speed_emb.T,
         jnp.pad(road_option_emb.T,
                 ((0, 0), (0, _TSEG - MAX_ROAD_OPTIONS)))],
        axis=1)
    out = _sc_lookup(mcols, table_t, jnp.asarray(_BINS)).T
    return (out, out)


# speedup vs baseline: 2.3739x; 1.1154x over previous
import functools

import numpy as np

import jax
import jax.numpy as jnp
from jax import lax
from jax.experimental import pallas as pl
from jax.experimental.pallas import tpu as pltpu
from jax.experimental.pallas import tpu_sc as plsc

TARGET_DISC = 1000
SPEED_DISC = 1000
MAX_ROAD_OPTIONS = 10
EMB_DIM = 8
BATCH = 16384

_INFO = plsc.get_sparse_core_info()
_NC, _NS, _L = _INFO.num_cores, _INFO.num_subcores, _INFO.num_lanes
_NW = _NC * _NS
_NROWS = 2 * TARGET_DISC + SPEED_DISC + MAX_ROAD_OPTIONS
_NPAD = 3016
_ROAD_WIN = _NPAD - 1000
_NCOMP = 4
_NGRP = _NW // _NCOMP
_BPW = BATCH // _NGRP
_NVEC = _BPW // _L
_TSEG = 1000


def _f32_linspace(start, stop, num):
    inv = np.float32(np.float32(1.0) / np.float32(num - 1))
    i = np.arange(num - 1, dtype=np.float32)
    t = i * inv
    head = np.float32(start) * (np.float32(1.0) - t) \
        + i * (np.float32(stop) * inv)
    return np.concatenate([head, np.array([stop], np.float32)])


_BINS = np.stack([
    _f32_linspace(-0.001, 0.001, TARGET_DISC),
    _f32_linspace(-0.001, 0.001, TARGET_DISC),
    _f32_linspace(-60.0, 60.0, SPEED_DISC),
    np.zeros(TARGET_DISC, np.float32),
])


def _sc_body(mc_hbm, tab_hbm, bins_hbm, out_hbm,
             m_v, tab_v, bins_v, rows_v, sem):
    wid = lax.axis_index("s") * _NC + lax.axis_index("c")
    comp = lax.rem(wid, _NCOMP)
    base = lax.div(wid, _NCOMP) * _BPW
    is_road = comp == _NCOMP - 1

    with jax.named_scope("in_dma"):
        copies = [
            pltpu.async_copy(mc_hbm.at[comp, pl.ds(base, _BPW)], m_v, sem),
            pltpu.async_copy(bins_hbm.at[comp], bins_v, sem),
        ]
        coff = jnp.where(is_road, _ROAD_WIN, comp * _TSEG)
        for k in range(4):
            p = lax.rem(k + wid, 4) * 2
            copies.append(pltpu.async_copy(
                tab_hbm.at[pl.ds(p, 2), pl.ds(coff, _TSEG)],
                tab_v.at[pl.ds(p, 2)], sem))
        for c in copies:
            c.wait()

    lo = jnp.where(comp < 2, jnp.float32(-0.001),
                   jnp.where(comp == 2, jnp.float32(-60.0), jnp.float32(0.0)))
    inv_step = jnp.where(comp < 2, jnp.float32(499500.0),
                         jnp.where(comp == 2, jnp.float32(8.325),
                                   jnp.float32(1.0)))
    hi_clip = jnp.where(is_road, MAX_ROAD_OPTIONS - 1, _TSEG - 1)
    road16 = jnp.broadcast_to(is_road, (_L,))
    loc_off = jnp.where(is_road, (_NCOMP - 1) * _TSEG - _ROAD_WIN, 0)
    nb = _TSEG

    with jax.named_scope("idx_gather"):
        @plsc.parallel_loop(0, _NVEC, unroll=4)
        def _work(i):
            x = m_v[pl.ds(i * _L, _L)]
            t = jnp.clip((x - lo) * inv_step, -1.0, float(nb)) + 1.0
            g = jnp.clip(lax.convert_element_type(t, jnp.int32), 0, nb)
            bin_hi = plsc.load_gather(bins_v, [jnp.clip(g, 0, nb - 1)])
            bin_lo = plsc.load_gather(bins_v, [jnp.clip(g - 1, 0, nb - 1)])
            one = jnp.full((_L,), 1, jnp.int32)
            zero = jnp.full((_L,), 0, jnp.int32)
            up = jnp.where((g < nb) & (bin_hi <= x), one, zero)
            dn = jnp.where((g > 0) & (bin_lo > x), one, zero)
            srch = g + up - dn
            road = lax.convert_element_type(x, jnp.int32)
            rid = jnp.clip(jnp.where(road16, road, srch), 0, hi_clip) + loc_off
            sl = pl.ds(i * _L, _L)
            for e in range(EMB_DIM):
                row = jnp.full((_L,), e, jnp.int32)
                rows_v[e, sl] = plsc.load_gather(tab_v, [row, rid])

    with jax.named_scope("out_dma"):
        pltpu.sync_copy(
            rows_v,
            out_hbm.at[pl.ds(comp * EMB_DIM, EMB_DIM), pl.ds(base, _BPW)])


_sc_lookup = functools.partial(
    pl.kernel,
    out_type=jax.ShapeDtypeStruct((_NCOMP * EMB_DIM, BATCH), jnp.float32),
    mesh=plsc.VectorSubcoreMesh(core_axis_name="c", subcore_axis_name="s"),
    compiler_params=pltpu.CompilerParams(
        needs_layout_passes=False, use_tc_tiling_on_sc=False),
    scratch_types=[
        pltpu.VMEM((_BPW,), jnp.float32),
        pltpu.VMEM((EMB_DIM, _TSEG), jnp.float32),
        pltpu.VMEM((_TSEG,), jnp.float32),
        pltpu.VMEM((EMB_DIM, _BPW), jnp.float32),
        pltpu.SemaphoreType.DMA,
    ],
)(_sc_body)


def kernel(metrics, target_x_emb, target_y_emb, speed_emb, road_option_emb):
    mcols = jnp.stack([metrics[:, 0], metrics[:, 1],
                       metrics[:, 2], metrics[:, 3]])
    table_t = jnp.concatenate(
        [target_x_emb.T, target_y_emb.T, speed_emb.T, road_option_emb.T,
         jnp.zeros((EMB_DIM, _NPAD - _NROWS), jnp.float32)], axis=1)
    out = _sc_lookup(mcols, table_t, jnp.asarray(_BINS)).T
    return (out, out)


# speedup vs baseline: 2.4937x; 1.0505x over previous
import functools

import numpy as np

import jax
import jax.numpy as jnp
from jax import lax
from jax.experimental import pallas as pl
from jax.experimental.pallas import tpu as pltpu
from jax.experimental.pallas import tpu_sc as plsc

TARGET_DISC = 1000
SPEED_DISC = 1000
MAX_ROAD_OPTIONS = 10
EMB_DIM = 8
BATCH = 16384

_INFO = plsc.get_sparse_core_info()
_NC, _NS, _L = _INFO.num_cores, _INFO.num_subcores, _INFO.num_lanes
_NW = _NC * _NS
_NROWS = 2 * TARGET_DISC + SPEED_DISC + MAX_ROAD_OPTIONS
_NPAD = 3016
_ROAD_WIN = _NPAD - 1000
_NCOMP = 4
_NGRP = _NW // _NCOMP
_BPW = BATCH // _NGRP
_NVEC = _BPW // _L
_TSEG = 1000


def _f32_linspace(start, stop, num):
    inv = np.float32(np.float32(1.0) / np.float32(num - 1))
    i = np.arange(num - 1, dtype=np.float32)
    t = i * inv
    head = np.float32(start) * (np.float32(1.0) - t) \
        + i * (np.float32(stop) * inv)
    return np.concatenate([head, np.array([stop], np.float32)])


_BINS = np.stack([
    _f32_linspace(-0.001, 0.001, TARGET_DISC),
    _f32_linspace(-0.001, 0.001, TARGET_DISC),
    _f32_linspace(-60.0, 60.0, SPEED_DISC),
    np.zeros(TARGET_DISC, np.float32),
])


def _sc_body(mc_hbm, tab_hbm, bins_hbm, out_hbm,
             m_v, tab_v, bins_v, rows_v, sem):
    wid = lax.axis_index("s") * _NC + lax.axis_index("c")
    comp = lax.rem(wid, _NCOMP)
    base = lax.div(wid, _NCOMP) * _BPW
    is_road = comp == _NCOMP - 1

    with jax.named_scope("in_dma"):
        copies = [
            pltpu.async_copy(mc_hbm.at[comp, pl.ds(base, _BPW)], m_v, sem),
            pltpu.async_copy(bins_hbm.at[comp], bins_v, sem),
        ]
        coff = jnp.where(is_road, _ROAD_WIN, comp * _TSEG)
        for k in range(4):
            p = lax.rem(k + wid, 4) * 2
            copies.append(pltpu.async_copy(
                tab_hbm.at[pl.ds(p, 2), pl.ds(coff, _TSEG)],
                tab_v.at[pl.ds(p, 2)], sem))
        for c in copies:
            c.wait()

    lo = jnp.where(comp < 2, jnp.float32(-0.001),
                   jnp.where(comp == 2, jnp.float32(-60.0), jnp.float32(0.0)))
    inv_step = jnp.where(comp < 2, jnp.float32(499500.0),
                         jnp.where(comp == 2, jnp.float32(8.325),
                                   jnp.float32(1.0)))
    hi_clip = jnp.where(is_road, MAX_ROAD_OPTIONS - 1, _TSEG - 1)
    road16 = jnp.broadcast_to(is_road, (_L,))
    loc_off = jnp.where(is_road, (_NCOMP - 1) * _TSEG - _ROAD_WIN, 0)
    nb = _TSEG

    with jax.named_scope("idx_gather"):
        @plsc.parallel_loop(0, _NVEC, unroll=4)
        def _work(i):
            x = m_v[pl.ds(i * _L, _L)]
            t = jnp.clip((x - lo) * inv_step, -1.0, float(nb)) + 1.0
            g = jnp.clip(lax.convert_element_type(t, jnp.int32), 0, nb)
            bin_hi = plsc.load_gather(bins_v, [jnp.clip(g, 0, nb - 1)])
            bin_lo = plsc.load_gather(bins_v, [jnp.clip(g - 1, 0, nb - 1)])
            one = jnp.full((_L,), 1, jnp.int32)
            zero = jnp.full((_L,), 0, jnp.int32)
            up = jnp.where((g < nb) & (bin_hi <= x), one, zero)
            dn = jnp.where((g > 0) & (bin_lo > x), one, zero)
            srch = g + up - dn
            road = lax.convert_element_type(x, jnp.int32)
            rid = jnp.clip(jnp.where(road16, road, srch), 0, hi_clip) + loc_off
            sl = pl.ds(i * _L, _L)
            for e in range(EMB_DIM):
                row = jnp.full((_L,), e, jnp.int32)
                rows_v[e, sl] = plsc.load_gather(tab_v, [row, rid])

    with jax.named_scope("out_dma"):
        pltpu.sync_copy(
            rows_v,
            out_hbm.at[pl.ds(comp * EMB_DIM, EMB_DIM), pl.ds(base, _BPW)])


_sc_lookup = functools.partial(
    pl.kernel,
    out_type=jax.ShapeDtypeStruct((_NCOMP * EMB_DIM, BATCH), jnp.float32),
    mesh=plsc.VectorSubcoreMesh(core_axis_name="c", subcore_axis_name="s"),
    compiler_params=pltpu.CompilerParams(
        needs_layout_passes=False, use_tc_tiling_on_sc=False),
    scratch_types=[
        pltpu.VMEM((_BPW,), jnp.float32),
        pltpu.VMEM((EMB_DIM, _TSEG), jnp.float32),
        pltpu.VMEM((_TSEG,), jnp.float32),
        pltpu.VMEM((EMB_DIM, _BPW), jnp.float32),
        pltpu.SemaphoreType.DMA,
    ],
)(_sc_body)


def kernel(metrics, target_x_emb, target_y_emb, speed_emb, road_option_emb):
    mcols = metrics.T
    table_t = jnp.concatenate(
        [target_x_emb.T, target_y_emb.T, speed_emb.T, road_option_emb.T,
         jnp.zeros((EMB_DIM, _NPAD - _NROWS), jnp.float32)], axis=1)
    out = _sc_lookup(mcols, table_t, jnp.asarray(_BINS)).T
    return (out, out)
